# Initial kernel scaffold; baseline (speedup 1.0000x reference)
import jax
import jax.numpy as jnp
from jax.experimental import pallas as pl


def kernel(x, edge_index, batch, W_in, b_in, eps, W1, b1, gamma, beta, W2, b2, Wo1, bo1, Wo2, bo2):
    raise NotImplementedError("write your pallas kernel here")


# speedup vs baseline: 4.6339x; 4.6339x over previous
import functools

import jax
import jax.numpy as jnp
from jax import lax
from jax.experimental import pallas as pl
from jax.experimental.pallas import tpu as pltpu
from jax.experimental.pallas import tpu_sc as plsc

N = 10000
E = 320000
D = 128
G = 64

_NC = 2
_NS = 16
_NW = _NC * _NS
_EPW = E // _NW
_CHUNK = 80
_NCHUNK = _EPW // _CHUNK
_NPAD = 10240
_RPT = _NPAD // _NS
_ZROWS = 128


def _sc_agg_body(r_hbm, src_hbm, dst_hbm, out_hbm,
                 src_v, dst_v, rows_v, zbuf, agg_sh, sem):
    cid = lax.axis_index("c")
    sid = lax.axis_index("s")
    wid = sid * _NC + cid

    zero16 = jnp.zeros((16,), jnp.float32)

    def zrow(i, carry):
        for j in range(D // 16):
            zbuf[i, pl.ds(j * 16, 16)] = zero16
        return carry

    lax.fori_loop(0, _ZROWS, zrow, 0)

    row0 = sid * _RPT
    for t in range(_RPT // _ZROWS):
        pltpu.sync_copy(zbuf, agg_sh.at[pl.ds(row0 + t * _ZROWS, _ZROWS)])
    plsc.subcore_barrier()

    def body(g, carry):
        base = pl.multiple_of(wid * _EPW + g * _CHUNK, _CHUNK)
        pltpu.sync_copy(src_hbm.at[pl.ds(base, _CHUNK)], src_v)
        pltpu.sync_copy(dst_hbm.at[pl.ds(base, _CHUNK)], dst_v)
        pltpu.async_copy(r_hbm.at[src_v], rows_v, sem).wait()
        pltpu.sync_copy(rows_v, agg_sh.at[dst_v], add=True)
        return carry

    lax.fori_loop(0, _NCHUNK, body, 0)

    plsc.subcore_barrier()
    pltpu.sync_copy(agg_sh.at[pl.ds(row0, _RPT)],
                    out_hbm.at[cid, pl.ds(row0, _RPT)])


_sc_agg = functools.partial(
    pl.kernel,
    mesh=plsc.VectorSubcoreMesh(core_axis_name="c", subcore_axis_name="s"),
    out_type=jax.ShapeDtypeStruct((_NC, _NPAD, D), jnp.float32),
    scratch_types=[
        pltpu.VMEM((_CHUNK,), jnp.int32),
        pltpu.VMEM((_CHUNK,), jnp.int32),
        pltpu.VMEM((_CHUNK, D), jnp.float32),
        pltpu.VMEM((_ZROWS, D), jnp.float32),
        pltpu.VMEM_SHARED((_NPAD, D), jnp.float32),
        pltpu.SemaphoreType.DMA,
    ],
)(_sc_agg_body)


def _in_body(x_ref, w_ref, b_ref, h_ref, r_ref):
    h = jnp.dot(x_ref[...], w_ref[...],
                preferred_element_type=jnp.float32) + b_ref[...]
    h_ref[...] = h
    r_ref[...] = jnp.maximum(h, 0.0)


_in_call = pl.pallas_call(
    _in_body,
    out_shape=[jax.ShapeDtypeStruct((_NPAD, D), jnp.float32),
               jax.ShapeDtypeStruct((_NPAD, D), jnp.float32)],
)


_MLP_BLK = 1280


def _mlp_body(s_ref, h_ref, a_ref, w1_ref, b1_ref, g_ref, be_ref,
              w2_ref, b2_ref, ho_ref, ro_ref):
    h = h_ref[...]
    z = s_ref[0] * h + a_ref[0] + a_ref[1]
    t = jnp.dot(z, w1_ref[...], preferred_element_type=jnp.float32) + b1_ref[...]
    mu = jnp.mean(t, axis=-1, keepdims=True)
    c = t - mu
    var = jnp.mean(c * c, axis=-1, keepdims=True)
    t = c * lax.rsqrt(var + 1e-5) * g_ref[...] + be_ref[...]
    t = jnp.maximum(t, 0.0)
    u = jnp.dot(t, w2_ref[...], preferred_element_type=jnp.float32) + b2_ref[...]
    hn = h + u
    ho_ref[...] = hn
    ro_ref[...] = jnp.maximum(hn, 0.0)


_mlp_call = pl.pallas_call(
    _mlp_body,
    grid=(_NPAD // _MLP_BLK,),
    in_specs=[
        pl.BlockSpec(memory_space=pltpu.SMEM),
        pl.BlockSpec((_MLP_BLK, D), lambda i: (i, 0)),
        pl.BlockSpec((_NC, _MLP_BLK, D), lambda i: (0, i, 0)),
        pl.BlockSpec((D, 2 * D), lambda i: (0, 0)),
        pl.BlockSpec((1, 2 * D), lambda i: (0, 0)),
        pl.BlockSpec((1, 2 * D), lambda i: (0, 0)),
        pl.BlockSpec((1, 2 * D), lambda i: (0, 0)),
        pl.BlockSpec((2 * D, D), lambda i: (0, 0)),
        pl.BlockSpec((1, D), lambda i: (0, 0)),
    ],
    out_specs=[
        pl.BlockSpec((_MLP_BLK, D), lambda i: (i, 0)),
        pl.BlockSpec((_MLP_BLK, D), lambda i: (i, 0)),
    ],
    out_shape=[jax.ShapeDtypeStruct((_NPAD, D), jnp.float32),
               jax.ShapeDtypeStruct((_NPAD, D), jnp.float32)],
)


def _head_body(b_ref, h_ref, wo1_ref, bo1_ref, wo2_ref, bo2_ref, o_ref):
    seg = b_ref[...]
    gid = lax.broadcasted_iota(jnp.int32, (G, _NPAD), 0)
    onehot = jnp.where(gid == seg, 1.0, 0.0)
    pooled = jnp.dot(onehot, h_ref[...], preferred_element_type=jnp.float32)
    t = jnp.dot(pooled, wo1_ref[...], preferred_element_type=jnp.float32)
    t = jnp.maximum(t + bo1_ref[...], 0.0)
    o_ref[...] = jnp.dot(t, wo2_ref[...],
                         preferred_element_type=jnp.float32) + bo2_ref[...]


_head_call = pl.pallas_call(
    _head_body,
    out_shape=jax.ShapeDtypeStruct((G, D), jnp.float32),
)


def kernel(x, edge_index, batch, W_in, b_in, eps, W1, b1, gamma, beta,
           W2, b2, Wo1, bo1, Wo2, bo2):
    src = edge_index[0].astype(jnp.int32)
    dst = edge_index[1].astype(jnp.int32)
    x_pad = jnp.zeros((_NPAD, D), jnp.float32).at[:N].set(x)
    batch_pad = jnp.concatenate(
        [batch.astype(jnp.int32), jnp.full((_NPAD - N,), G, jnp.int32)]
    ).reshape(1, _NPAD)

    h, r = _in_call(x_pad, W_in, b_in.reshape(1, D))
    for i in range(3):
        agg = _sc_agg(r, src, dst)
        scale = (1.0 + eps[i]).reshape(1)
        h, r = _mlp_call(scale, h, agg, W1[i], b1[i].reshape(1, 2 * D),
                         gamma[i].reshape(1, 2 * D), beta[i].reshape(1, 2 * D),
                         W2[i], b2[i].reshape(1, D))
    out = _head_call(batch_pad, h, Wo1, bo1.reshape(1, 2 * D),
                     Wo2, bo2.reshape(1, D))
    return out.reshape(-1)


# speedup vs baseline: 10.2667x; 2.2156x over previous
import functools

import jax
import jax.numpy as jnp
from jax import lax
from jax.experimental import pallas as pl
from jax.experimental.pallas import tpu as pltpu
from jax.experimental.pallas import tpu_sc as plsc

N = 10000
E = 320000
D = 128
G = 64

_NC = 2
_NS = 16
_NW = _NC * _NS
_EPW = E // _NW
_CHUNK = 80
_NCHUNK = _EPW // _CHUNK
_NPAD = 10240
_RPT = _NPAD // _NS
_SLAB = 25
_NSLAB = _NCHUNK // _SLAB


def _sc_agg_body(r_hbm, src_hbm, dst_hbm, out_hbm,
                 sidx0, didx0, sidx1, didx1, rows0, rows1, agg_sh,
                 isem0, isem1, gsem0, gsem1):
    cid = lax.axis_index("c")
    sid = lax.axis_index("s")
    wid = sid * _NC + cid

    zero16 = jnp.zeros((16,), jnp.float32)

    def zrow(i, carry):
        for j in range(D // 16):
            rows0[i, pl.ds(j * 16, 16)] = zero16
        return carry

    lax.fori_loop(0, _CHUNK, zrow, 0)

    row0 = sid * _RPT
    for t in range(_RPT // _CHUNK):
        pltpu.sync_copy(rows0, agg_sh.at[pl.ds(row0 + t * _CHUNK, _CHUNK)])
    plsc.subcore_barrier()

    ibufs = [(sidx0, didx0, isem0), (sidx1, didx1, isem1)]
    rbufs = [(rows0, gsem0), (rows1, gsem1)]

    def gather(sa, g, buf, sem):
        return pltpu.async_copy(r_hbm.at[sa.at[g]], buf, sem)

    def scatter(da, g, buf):
        pltpu.sync_copy(buf, agg_sh.at[da.at[g]], add=True)

    pltpu.sync_copy(src_hbm.at[wid, 0], sidx0)
    pltpu.sync_copy(dst_hbm.at[wid, 0], didx0)

    for s in range(_NSLAB):
        sa, da, _ = ibufs[s % 2]
        if s + 1 < _NSLAB:
            sb, db, isem_n = ibufs[(s + 1) % 2]
            pltpu.async_copy(src_hbm.at[wid, s + 1], sb, isem_n)
            pltpu.async_copy(dst_hbm.at[wid, s + 1], db, isem_n)

        gather(sa, 0, rows0, gsem0)

        def body(k, carry):
            g0 = 2 * k
            g1 = g0 + 1
            gather(sa, g1, rows1, gsem1)
            pltpu.make_async_copy(r_hbm.at[sa.at[g0]], rows0, gsem0).wait()
            scatter(da, g0, rows0)

            @pl.when(g1 + 1 < _SLAB)
            def _():
                gather(sa, g1 + 1, rows0, gsem0)

            pltpu.make_async_copy(r_hbm.at[sa.at[g1]], rows1, gsem1).wait()
            scatter(da, g1, rows1)
            return carry

        lax.fori_loop(0, _SLAB // 2, body, 0)
        if _SLAB % 2:
            g_last = _SLAB - 1
            pltpu.make_async_copy(r_hbm.at[sa.at[g_last]], rows0, gsem0).wait()
            scatter(da, g_last, rows0)

        if s + 1 < _NSLAB:
            pltpu.make_async_copy(src_hbm.at[wid, s + 1], sb, isem_n).wait()
            pltpu.make_async_copy(dst_hbm.at[wid, s + 1], db, isem_n).wait()

    plsc.subcore_barrier()
    pltpu.sync_copy(agg_sh.at[pl.ds(row0, _RPT)],
                    out_hbm.at[cid, pl.ds(row0, _RPT)])


_sc_agg = functools.partial(
    pl.kernel,
    mesh=plsc.VectorSubcoreMesh(core_axis_name="c", subcore_axis_name="s"),
    out_type=jax.ShapeDtypeStruct((_NC, _NPAD, D), jnp.float32),
    scratch_types=[
        pltpu.VMEM((_SLAB, _CHUNK), jnp.int32),
        pltpu.VMEM((_SLAB, _CHUNK), jnp.int32),
        pltpu.VMEM((_SLAB, _CHUNK), jnp.int32),
        pltpu.VMEM((_SLAB, _CHUNK), jnp.int32),
        pltpu.VMEM((_CHUNK, D), jnp.float32),
        pltpu.VMEM((_CHUNK, D), jnp.float32),
        pltpu.VMEM_SHARED((_NPAD, D), jnp.float32),
        pltpu.SemaphoreType.DMA,
        pltpu.SemaphoreType.DMA,
        pltpu.SemaphoreType.DMA,
        pltpu.SemaphoreType.DMA,
    ],
)(_sc_agg_body)


def _in_body(x_ref, w_ref, b_ref, h_ref, r_ref):
    h = jnp.dot(x_ref[...], w_ref[...],
                preferred_element_type=jnp.float32) + b_ref[...]
    h_ref[...] = h
    r_ref[...] = jnp.maximum(h, 0.0)


_in_call = pl.pallas_call(
    _in_body,
    out_shape=[jax.ShapeDtypeStruct((_NPAD, D), jnp.float32),
               jax.ShapeDtypeStruct((_NPAD, D), jnp.float32)],
)


_MLP_BLK = 1280


def _mlp_body(s_ref, h_ref, a_ref, w1_ref, b1_ref, g_ref, be_ref,
              w2_ref, b2_ref, ho_ref, ro_ref):
    h = h_ref[...]
    z = s_ref[0] * h + a_ref[0] + a_ref[1]
    t = jnp.dot(z, w1_ref[...], preferred_element_type=jnp.float32) + b1_ref[...]
    mu = jnp.mean(t, axis=-1, keepdims=True)
    c = t - mu
    var = jnp.mean(c * c, axis=-1, keepdims=True)
    t = c * lax.rsqrt(var + 1e-5) * g_ref[...] + be_ref[...]
    t = jnp.maximum(t, 0.0)
    u = jnp.dot(t, w2_ref[...], preferred_element_type=jnp.float32) + b2_ref[...]
    hn = h + u
    ho_ref[...] = hn
    ro_ref[...] = jnp.maximum(hn, 0.0)


_mlp_call = pl.pallas_call(
    _mlp_body,
    grid=(_NPAD // _MLP_BLK,),
    in_specs=[
        pl.BlockSpec(memory_space=pltpu.SMEM),
        pl.BlockSpec((_MLP_BLK, D), lambda i: (i, 0)),
        pl.BlockSpec((_NC, _MLP_BLK, D), lambda i: (0, i, 0)),
        pl.BlockSpec((D, 2 * D), lambda i: (0, 0)),
        pl.BlockSpec((1, 2 * D), lambda i: (0, 0)),
        pl.BlockSpec((1, 2 * D), lambda i: (0, 0)),
        pl.BlockSpec((1, 2 * D), lambda i: (0, 0)),
        pl.BlockSpec((2 * D, D), lambda i: (0, 0)),
        pl.BlockSpec((1, D), lambda i: (0, 0)),
    ],
    out_specs=[
        pl.BlockSpec((_MLP_BLK, D), lambda i: (i, 0)),
        pl.BlockSpec((_MLP_BLK, D), lambda i: (i, 0)),
    ],
    out_shape=[jax.ShapeDtypeStruct((_NPAD, D), jnp.float32),
               jax.ShapeDtypeStruct((_NPAD, D), jnp.float32)],
)


def _head_body(b_ref, h_ref, wo1_ref, bo1_ref, wo2_ref, bo2_ref, o_ref):
    seg = b_ref[...]
    gid = lax.broadcasted_iota(jnp.int32, (G, _NPAD), 0)
    onehot = jnp.where(gid == seg, 1.0, 0.0)
    pooled = jnp.dot(onehot, h_ref[...], preferred_element_type=jnp.float32)
    t = jnp.dot(pooled, wo1_ref[...], preferred_element_type=jnp.float32)
    t = jnp.maximum(t + bo1_ref[...], 0.0)
    o_ref[...] = jnp.dot(t, wo2_ref[...],
                         preferred_element_type=jnp.float32) + bo2_ref[...]


_head_call = pl.pallas_call(
    _head_body,
    out_shape=jax.ShapeDtypeStruct((G, D), jnp.float32),
)


def kernel(x, edge_index, batch, W_in, b_in, eps, W1, b1, gamma, beta,
           W2, b2, Wo1, bo1, Wo2, bo2):
    src = edge_index[0].astype(jnp.int32).reshape(_NW, _NSLAB, _SLAB, _CHUNK)
    dst = edge_index[1].astype(jnp.int32).reshape(_NW, _NSLAB, _SLAB, _CHUNK)
    x_pad = jnp.zeros((_NPAD, D), jnp.float32).at[:N].set(x)
    batch_pad = jnp.concatenate(
        [batch.astype(jnp.int32), jnp.full((_NPAD - N,), G, jnp.int32)]
    ).reshape(1, _NPAD)

    h, r = _in_call(x_pad, W_in, b_in.reshape(1, D))
    for i in range(3):
        agg = _sc_agg(r, src, dst)
        scale = (1.0 + eps[i]).reshape(1)
        h, r = _mlp_call(scale, h, agg, W1[i], b1[i].reshape(1, 2 * D),
                         gamma[i].reshape(1, 2 * D), beta[i].reshape(1, 2 * D),
                         W2[i], b2[i].reshape(1, D))
    out = _head_call(batch_pad, h, Wo1, bo1.reshape(1, 2 * D),
                     Wo2, bo2.reshape(1, D))
    return out.reshape(-1)
